# baseline (speedup 1.0000x reference)
import jax
import jax.numpy as jnp
from jax.experimental import pallas as pl
from jax.experimental.pallas import tpu as pltpu

_CH = 1024
_NBUF = 4


def _router_body(x_hbm, w1_ref, b1_ref, w2_ref, b2_ref, o_ref, xbuf, sems):
    M = x_hbm.shape[0]
    nchunks = M // _CH
    w1 = w1_ref[...]
    b1 = b1_ref[...]
    w2 = w2_ref[...]
    b2 = b2_ref[...]

    def copy(j):
        slot = j % _NBUF
        return pltpu.make_async_copy(
            x_hbm.at[pl.ds(j * _CH, _CH), :], xbuf.at[slot], sems.at[slot]
        )

    for j in range(_NBUF - 1):
        copy(j).start()

    for j in range(nchunks):
        nxt = j + _NBUF - 1
        if nxt < nchunks:
            copy(nxt).start()
        copy(j).wait()
        xb = xbuf[j % _NBUF].astype(jnp.bfloat16)
        h = jnp.dot(xb, w1, preferred_element_type=jnp.float32)
        h = jnp.maximum(h + b1, 0.0)
        logits = jnp.dot(h, w2, preferred_element_type=jnp.float32) + b2
        m = jnp.max(logits, axis=-1, keepdims=True)
        e = jnp.exp(logits - m)
        o_ref[pl.ds(j * _CH, _CH), :] = e / jnp.sum(e, axis=-1, keepdims=True)


def kernel(x, W1, b1, W2, b2):
    M, K = x.shape
    N1 = W1.shape[1]
    N2 = W2.shape[1]

    W1b = W1.astype(jnp.bfloat16)
    b1r = b1.reshape(1, N1)
    b2r = b2.reshape(1, N2)

    return pl.pallas_call(
        _router_body,
        in_specs=[
            pl.BlockSpec(memory_space=pltpu.HBM),
            pl.BlockSpec((K, N1), lambda: (0, 0)),
            pl.BlockSpec((1, N1), lambda: (0, 0)),
            pl.BlockSpec((N1, N2), lambda: (0, 0)),
            pl.BlockSpec((1, N2), lambda: (0, 0)),
        ],
        out_specs=pl.BlockSpec((M, N2), lambda: (0, 0)),
        out_shape=jax.ShapeDtypeStruct((M, N2), jnp.float32),
        scratch_shapes=[
            pltpu.VMEM((_NBUF, _CH, K), jnp.float32),
            pltpu.SemaphoreType.DMA((_NBUF,)),
        ],
    )(x, W1b, b1r, W2, b2r)


# speedup vs baseline: 1.1154x; 1.1154x over previous
import jax
import jax.numpy as jnp
from jax.experimental import pallas as pl
from jax.experimental.pallas import tpu as pltpu


def _router_body(x_ref, w1_ref, b1_ref, w2_ref, b2_ref, o_ref):
    h = jnp.dot(x_ref[...], w1_ref[...], preferred_element_type=jnp.float32)
    h = jnp.maximum(h + b1_ref[...], 0.0)
    logits = jnp.dot(h, w2_ref[...], preferred_element_type=jnp.float32)
    logits = logits + b2_ref[...]
    m = jnp.max(logits, axis=-1, keepdims=True)
    e = jnp.exp(logits - m)
    o_ref[...] = e / jnp.sum(e, axis=-1, keepdims=True)


def kernel(x, W1, b1, W2, b2):
    M, K = x.shape
    N1 = W1.shape[1]
    N2 = W2.shape[1]
    BM = 2048

    b1r = b1.reshape(1, N1)
    b2r = b2.reshape(1, N2)

    return pl.pallas_call(
        _router_body,
        grid=(M // BM,),
        in_specs=[
            pl.BlockSpec((BM, K), lambda i: (i, 0)),
            pl.BlockSpec((K, N1), lambda i: (0, 0)),
            pl.BlockSpec((1, N1), lambda i: (0, 0)),
            pl.BlockSpec((N1, N2), lambda i: (0, 0)),
            pl.BlockSpec((1, N2), lambda i: (0, 0)),
        ],
        out_specs=pl.BlockSpec((BM, N2), lambda i: (i, 0)),
        out_shape=jax.ShapeDtypeStruct((M, N2), jnp.float32),
        compiler_params=pltpu.CompilerParams(
            dimension_semantics=("parallel",),
        ),
    )(x, W1, b1r, W2, b2r)


# speedup vs baseline: 1.1206x; 1.0047x over previous
import jax
import jax.numpy as jnp
from jax.experimental import pallas as pl
from jax.experimental.pallas import tpu as pltpu


def _router_body(x_ref, w1_ref, b1_ref, w2_ref, b2_ref, o_ref):
    h = jnp.dot(x_ref[...], w1_ref[...], preferred_element_type=jnp.float32)
    h = jnp.maximum(h + b1_ref[...], 0.0)
    logits = jnp.dot(h, w2_ref[...], preferred_element_type=jnp.float32)
    logits = logits + b2_ref[...]
    m = jnp.max(logits, axis=-1, keepdims=True)
    e = jnp.exp(logits - m)
    o_ref[...] = e / jnp.sum(e, axis=-1, keepdims=True)


def kernel(x, W1, b1, W2, b2):
    M, K = x.shape
    N1 = W1.shape[1]
    N2 = W2.shape[1]
    BM = 2048

    b1r = b1.reshape(1, N1)
    b2r = b2.reshape(1, N2)

    return pl.pallas_call(
        _router_body,
        grid=(M // BM,),
        in_specs=[
            pl.BlockSpec((BM, K), lambda i: (i, 0)),
            pl.BlockSpec((K, N1), lambda i: (0, 0)),
            pl.BlockSpec((1, N1), lambda i: (0, 0)),
            pl.BlockSpec((N1, N2), lambda i: (0, 0)),
            pl.BlockSpec((1, N2), lambda i: (0, 0)),
        ],
        out_specs=pl.BlockSpec((BM, N2), lambda i: (i, 0)),
        out_shape=jax.ShapeDtypeStruct((M, N2), jnp.float32),
        compiler_params=pltpu.CompilerParams(
            dimension_semantics=("arbitrary",),
        ),
    )(x, W1, b1r, W2, b2r)
